# Initial kernel scaffold; baseline (speedup 1.0000x reference)
import jax
import jax.numpy as jnp
from jax.experimental import pallas as pl


def kernel(x, edge_index, batch, resource_features, W1, att_src1, att_dst1, b1, W2, att_src2, att_dst2, b2, Wr, br, gamma, beta):
    raise NotImplementedError("write your pallas kernel here")


# speedup vs baseline: 1.0996x; 1.0996x over previous
import functools

import jax
import jax.numpy as jnp
from jax.experimental import pallas as pl
from jax.experimental.pallas import tpu as pltpu

N_NODES = 10000
D_FEAT = 128
HIDDEN = 128
HEADS = 2
N_RESOURCES = 16

_COMBINE_BLK = 400


def _combine_body(h2_ref, gamma_ref, beta_ref, r_ref, out_ref):
    h = h2_ref[...]
    mu = jnp.mean(h, axis=-1, keepdims=True)
    var = jnp.mean((h - mu) ** 2, axis=-1, keepdims=True)
    hn = (h - mu) / jnp.sqrt(var + 1e-5) * gamma_ref[...] + beta_ref[...]
    b = h.shape[0]
    out_ref[:, :, 0:HIDDEN] = jnp.broadcast_to(
        hn[:, None, :], (b, N_RESOURCES, HIDDEN))
    out_ref[:, :, HIDDEN:2 * HIDDEN] = jnp.broadcast_to(
        r_ref[...][None, :, :], (b, N_RESOURCES, HIDDEN))


def _combine(h2, gamma, beta, r):
    n = h2.shape[0]
    blk = _COMBINE_BLK
    grid = (n // blk,)
    return pl.pallas_call(
        _combine_body,
        grid=grid,
        in_specs=[
            pl.BlockSpec((blk, HIDDEN), lambda i: (i, 0)),
            pl.BlockSpec((1, HIDDEN), lambda i: (0, 0)),
            pl.BlockSpec((1, HIDDEN), lambda i: (0, 0)),
            pl.BlockSpec((N_RESOURCES, HIDDEN), lambda i: (0, 0)),
        ],
        out_specs=pl.BlockSpec((blk, N_RESOURCES, 2 * HIDDEN),
                               lambda i: (i, 0, 0)),
        out_shape=jax.ShapeDtypeStruct((n, N_RESOURCES, 2 * HIDDEN),
                                       jnp.float32),
    )(h2, gamma.reshape(1, HIDDEN), beta.reshape(1, HIDDEN), r)


def _gat_layer(x, src, dst, W, att_src, att_dst, bias, heads, out_ch, concat):
    n = x.shape[0]
    h = (x @ W).reshape(n, heads, out_ch)
    a_src = (h * att_src[None, :, :]).sum(-1)
    a_dst = (h * att_dst[None, :, :]).sum(-1)
    alpha = a_src[src] + a_dst[dst]
    alpha = jax.nn.leaky_relu(alpha, negative_slope=0.2)
    amax = jax.ops.segment_max(alpha, dst, num_segments=n)
    amax = jnp.where(jnp.isfinite(amax), amax, 0.0)
    ea = jnp.exp(alpha - amax[dst])
    denom = jax.ops.segment_sum(ea, dst, num_segments=n)
    msg = h[src] * ea[:, :, None]
    out = jax.ops.segment_sum(msg, dst, num_segments=n)
    out = out / denom[:, :, None]
    if concat:
        out = out.reshape(n, heads * out_ch)
    else:
        out = out.mean(axis=1)
    return out + bias


def kernel(x, edge_index, batch, resource_features,
           W1, att_src1, att_dst1, b1,
           W2, att_src2, att_dst2, b2,
           Wr, br, gamma, beta):
    n = x.shape[0]
    loops = jnp.arange(n, dtype=edge_index.dtype)
    src = jnp.concatenate([edge_index[0], loops])
    dst = jnp.concatenate([edge_index[1], loops])
    h1 = _gat_layer(x, src, dst, W1, att_src1, att_dst1, b1, HEADS, HIDDEN,
                    True)
    h1 = jax.nn.elu(h1)
    h2 = _gat_layer(h1, src, dst, W2, att_src2, att_dst2, b2, 1, HIDDEN,
                    False)
    r = jax.nn.elu(resource_features @ Wr + br)
    return _combine(h2, gamma, beta, r)


# speedup vs baseline: 31.9506x; 29.0555x over previous
import functools

import jax
import jax.numpy as jnp
from jax import lax
from jax.experimental import pallas as pl
from jax.experimental.pallas import tpu as pltpu
from jax.experimental.pallas import tpu_sc as plsc

N_NODES = 10000
D_FEAT = 128
HIDDEN = 128
HEADS = 2
N_RESOURCES = 16

NPAD = 10240
K = 128
E_RAW = 320000
E_LOOPS = E_RAW + N_NODES
EPAD = 331776
CH1 = EPAD // (16 * K)
CH2 = EPAD // (32 * K)
NT = 16
RPT = NPAD // NT

BLK_A = 640
BLK_C = 256


def _tc_a_body(x_ref, w1_ref, asrc_ref, adst_ref, htab_ref, a1_ref):
    xb = x_ref[...]
    hb = jnp.dot(xb, w1_ref[...], preferred_element_type=jnp.float32)
    arow = []
    for h in range(HEADS):
        hh = hb[:, h * HIDDEN:(h + 1) * HIDDEN]
        htab_ref[h] = hh
        arow.append(jnp.sum(hh * asrc_ref[h:h + 1, :], axis=1)[None, :])
        arow.append(jnp.sum(hh * adst_ref[h:h + 1, :], axis=1)[None, :])
    a1_ref[...] = jnp.concatenate(arow, axis=0)


def _tc_a(x_pad, W1, att_src1, att_dst1):
    grid = (NPAD // BLK_A,)
    return pl.pallas_call(
        _tc_a_body,
        grid=grid,
        in_specs=[
            pl.BlockSpec((BLK_A, D_FEAT), lambda i: (i, 0)),
            pl.BlockSpec((D_FEAT, HEADS * HIDDEN), lambda i: (0, 0)),
            pl.BlockSpec((HEADS, HIDDEN), lambda i: (0, 0)),
            pl.BlockSpec((HEADS, HIDDEN), lambda i: (0, 0)),
        ],
        out_specs=[
            pl.BlockSpec((HEADS, BLK_A, HIDDEN), lambda i: (0, i, 0)),
            pl.BlockSpec((2 * HEADS, BLK_A), lambda i: (0, i)),
        ],
        out_shape=[
            jax.ShapeDtypeStruct((HEADS, NPAD, HIDDEN), jnp.float32),
            jax.ShapeDtypeStruct((2 * HEADS, NPAD), jnp.float32),
        ],
    )(x_pad, W1, att_src1, att_dst1)


def _tc_b_body(p_ref, den_ref, w2_ref, b1_ref, as2_ref, ad2_ref,
               htab_ref, a2_ref):
    p0 = p_ref[0]
    p1 = p_ref[1]
    d0 = jnp.maximum(jnp.sum(den_ref[0], axis=0), 1e-30)[:, None]
    d1 = jnp.maximum(jnp.sum(den_ref[1], axis=0), 1e-30)[:, None]
    h1 = jnp.concatenate([p0 / d0, p1 / d1], axis=1) + b1_ref[...]
    h1 = jnp.where(h1 > 0, h1, jnp.exp(h1) - 1.0)
    hp = jnp.dot(h1, w2_ref[...], preferred_element_type=jnp.float32)
    htab_ref[...] = hp
    a2_ref[...] = jnp.concatenate(
        [jnp.sum(hp * as2_ref[...], axis=1)[None, :],
         jnp.sum(hp * ad2_ref[...], axis=1)[None, :]], axis=0)


def _tc_b(p1v, den1, W2, b1, att_src2, att_dst2):
    grid = (NPAD // BLK_A,)
    return pl.pallas_call(
        _tc_b_body,
        grid=grid,
        in_specs=[
            pl.BlockSpec((2, BLK_A, HIDDEN), lambda i: (0, i, 0)),
            pl.BlockSpec((2, NT, BLK_A), lambda i: (0, 0, i)),
            pl.BlockSpec((HEADS * HIDDEN, HIDDEN), lambda i: (0, 0)),
            pl.BlockSpec((1, HEADS * HIDDEN), lambda i: (0, 0)),
            pl.BlockSpec((1, HIDDEN), lambda i: (0, 0)),
            pl.BlockSpec((1, HIDDEN), lambda i: (0, 0)),
        ],
        out_specs=[
            pl.BlockSpec((BLK_A, HIDDEN), lambda i: (i, 0)),
            pl.BlockSpec((2, BLK_A), lambda i: (0, i)),
        ],
        out_shape=[
            jax.ShapeDtypeStruct((NPAD, HIDDEN), jnp.float32),
            jax.ShapeDtypeStruct((2, NPAD), jnp.float32),
        ],
    )(p1v, den1, W2, b1, att_src2, att_dst2)


def _tc_c_body(p_ref, den_ref, g_ref, be_ref, b2_ref, rf_ref, wr_ref,
               br_ref, out_ref):
    s = p_ref[0] + p_ref[1]
    d = jnp.maximum(jnp.sum(den_ref[0] + den_ref[1], axis=0),
                    1e-30)[:, None]
    h2 = s / d + b2_ref[...]
    mu = jnp.mean(h2, axis=1, keepdims=True)
    var = jnp.mean((h2 - mu) ** 2, axis=1, keepdims=True)
    h2 = (h2 - mu) / jnp.sqrt(var + 1e-5) * g_ref[...] + be_ref[...]
    r = jnp.dot(rf_ref[...], wr_ref[...],
                preferred_element_type=jnp.float32) + br_ref[...]
    r = jnp.where(r > 0, r, jnp.exp(r) - 1.0)
    out_ref[:, :, :HIDDEN] = jnp.broadcast_to(
        h2[:, None, :], (BLK_C, N_RESOURCES, HIDDEN))
    out_ref[:, :, HIDDEN:] = jnp.broadcast_to(
        r[None, :, :], (BLK_C, N_RESOURCES, HIDDEN))


def _tc_c(p2v, den2, gamma, beta, b2, rf, Wr, br):
    grid = (NPAD // BLK_C,)
    return pl.pallas_call(
        _tc_c_body,
        grid=grid,
        in_specs=[
            pl.BlockSpec((2, BLK_C, HIDDEN), lambda i: (0, i, 0)),
            pl.BlockSpec((2, NT, BLK_C), lambda i: (0, 0, i)),
            pl.BlockSpec((1, HIDDEN), lambda i: (0, 0)),
            pl.BlockSpec((1, HIDDEN), lambda i: (0, 0)),
            pl.BlockSpec((1, HIDDEN), lambda i: (0, 0)),
            pl.BlockSpec((N_RESOURCES, 32), lambda i: (0, 0)),
            pl.BlockSpec((32, HIDDEN), lambda i: (0, 0)),
            pl.BlockSpec((1, HIDDEN), lambda i: (0, 0)),
        ],
        out_specs=pl.BlockSpec((BLK_C, N_RESOURCES, 2 * HIDDEN),
                               lambda i: (i, 0, 0)),
        out_shape=jax.ShapeDtypeStruct((N_NODES, N_RESOURCES, 2 * HIDDEN),
                                       jnp.float32),
    )(p2v, den2, gamma, beta, b2, rf, Wr, br)


def _make_sc_pass(two_tables, chunks_per_tile):
    mesh = plsc.VectorSubcoreMesh(core_axis_name="c", subcore_axis_name="s")

    @functools.partial(
        pl.kernel, mesh=mesh,
        compiler_params=pltpu.CompilerParams(needs_layout_passes=False),
        out_type=[
            jax.ShapeDtypeStruct((2 * NPAD, HIDDEN), jnp.float32),
            jax.ShapeDtypeStruct((2 * NT * NPAD,), jnp.float32),
        ],
        scratch_types=[
            pltpu.VMEM((NPAD,), jnp.float32),
            pltpu.VMEM((NPAD,), jnp.float32),
            pltpu.VMEM((NPAD,), jnp.float32),
            pltpu.VMEM((K,), jnp.int32),
            pltpu.VMEM((K,), jnp.int32),
            pltpu.VMEM((K,), jnp.int32),
            pltpu.VMEM((K,), jnp.float32),
            pltpu.VMEM((K, HIDDEN), jnp.float32),
            pltpu.VMEM_SHARED((NPAD, HIDDEN), jnp.float32),
            pltpu.SemaphoreType.DMA,
        ])
    def sc_pass(htab, aflat, src_hbm, dst_hbm, zeros_hbm, zeros1_hbm,
                out_hbm, den_hbm,
                asrc_v, adst_v, denom_v, src_v, src2_v, dst_v, w_buf,
                rows_v, out_sh, sem):
        c = lax.axis_index("c")
        s = lax.axis_index("s")
        pltpu.sync_copy(zeros_hbm.at[pl.ds(s * RPT, RPT)],
                        out_sh.at[pl.ds(s * RPT, RPT)])
        pltpu.sync_copy(zeros1_hbm, denom_v)
        if two_tables:
            a_base = (2 * c) * NPAD
            tab_off = c * NPAD
            tile_base = s * chunks_per_tile * K
        else:
            a_base = 0
            tab_off = 0
            tile_base = (s * 2 + c) * chunks_per_tile * K
        pltpu.sync_copy(aflat.at[pl.ds(a_base, NPAD)], asrc_v)
        pltpu.sync_copy(aflat.at[pl.ds(a_base + NPAD, NPAD)], adst_v)
        plsc.subcore_barrier()

        def chunk(i, carry):
            base = tile_base + i * K
            pltpu.sync_copy(src_hbm.at[pl.ds(base, K)], src_v)
            pltpu.sync_copy(dst_hbm.at[pl.ds(base, K)], dst_v)

            def wblk(j, carry2):
                sl = pl.ds(j * 16, 16)
                si = src_v[sl]
                di = dst_v[sl]
                av = plsc.load_gather(asrc_v, [si])
                dv = plsc.load_gather(adst_v, [di])
                al = av + dv
                al = jnp.maximum(al, 0.0) + 0.2 * jnp.minimum(al, 0.0)
                w = jnp.exp(al)
                w_buf[sl] = w
                plsc.addupdate_scatter(denom_v, [di], w)
                if two_tables:
                    src2_v[sl] = si + tab_off
                return carry2

            lax.fori_loop(0, K // 16, wblk, 0)
            idx_ref = src2_v if two_tables else src_v
            pltpu.async_copy(htab.at[idx_ref], rows_v, sem).wait()

            def rblk(g, carry2):
                wg = w_buf[pl.ds(g * 16, 16)]
                for i in range(16):
                    wv = jnp.full((16,), wg[i], jnp.float32)
                    e = g * 16 + i
                    for j in range(HIDDEN // 16):
                        sl2 = pl.ds(j * 16, 16)
                        rows_v[e, sl2] = rows_v[e, sl2] * wv
                return carry2

            lax.fori_loop(0, K // 16, rblk, 0)
            pltpu.sync_copy(rows_v, out_sh.at[dst_v], add=True)
            return carry

        lax.fori_loop(0, chunks_per_tile, chunk, 0)
        pltpu.sync_copy(denom_v,
                        den_hbm.at[pl.ds((c * NT + s) * NPAD, NPAD)])
        plsc.subcore_barrier()
        pltpu.sync_copy(out_sh.at[pl.ds(s * RPT, RPT)],
                        out_hbm.at[pl.ds(c * NPAD + s * RPT, RPT)])

    return sc_pass


_make_sc_pass = functools.lru_cache(maxsize=None)(_make_sc_pass)


def kernel(x, edge_index, batch, resource_features,
           W1, att_src1, att_dst1, b1,
           W2, att_src2, att_dst2, b2,
           Wr, br, gamma, beta):
    n = x.shape[0]
    loops = jnp.arange(n, dtype=jnp.int32)
    fill = jnp.full((EPAD - E_LOOPS,), n, jnp.int32)
    src = jnp.concatenate([edge_index[0].astype(jnp.int32), loops, fill])
    dst = jnp.concatenate([edge_index[1].astype(jnp.int32), loops, fill])

    x_pad = jnp.zeros((NPAD, D_FEAT), jnp.float32).at[:n].set(x)
    zeros_tab = jnp.zeros((NPAD, HIDDEN), jnp.float32)
    zeros_vec = jnp.zeros((NPAD,), jnp.float32)

    _sc_layer1 = _make_sc_pass(True, CH1)
    _sc_layer2 = _make_sc_pass(False, CH2)

    htab1, a1 = _tc_a(x_pad, W1, att_src1, att_dst1)
    out1, den1 = _sc_layer1(htab1.reshape(HEADS * NPAD, HIDDEN),
                            a1.reshape(2 * HEADS * NPAD), src, dst,
                            zeros_tab, zeros_vec)
    htab2, a2 = _tc_b(out1.reshape(2, NPAD, HIDDEN),
                      den1.reshape(2, NT, NPAD), W2,
                      b1.reshape(1, HEADS * HIDDEN),
                      att_src2, att_dst2)
    out2, den2 = _sc_layer2(htab2, a2.reshape(2 * NPAD), src, dst,
                            zeros_tab, zeros_vec)
    return _tc_c(out2.reshape(2, NPAD, HIDDEN), den2.reshape(2, NT, NPAD),
                 gamma.reshape(1, HIDDEN), beta.reshape(1, HIDDEN),
                 b2.reshape(1, HIDDEN), resource_features, Wr,
                 br.reshape(1, HIDDEN))


# speedup vs baseline: 40.5801x; 1.2701x over previous
import functools

import jax
import jax.numpy as jnp
from jax import lax
from jax.experimental import pallas as pl
from jax.experimental.pallas import tpu as pltpu
from jax.experimental.pallas import tpu_sc as plsc

N_NODES = 10000
D_FEAT = 128
HIDDEN = 128
HEADS = 2
N_RESOURCES = 16

NPAD = 10240
K = 64
E_RAW = 320000
E_LOOPS = E_RAW + N_NODES
CH1 = 324
CH2 = 162
EPAD = 331776
NT = 16
RPT = NPAD // NT

BLK_A = 640
BLK_C = 256


def _tc_a_body(x_ref, w1_ref, asrc_ref, adst_ref, htab_ref, a1_ref):
    xb = x_ref[...]
    hb = jnp.dot(xb, w1_ref[...], preferred_element_type=jnp.float32)
    arow = []
    for h in range(HEADS):
        hh = hb[:, h * HIDDEN:(h + 1) * HIDDEN]
        htab_ref[h] = hh
        arow.append(jnp.sum(hh * asrc_ref[h:h + 1, :], axis=1)[None, :])
        arow.append(jnp.sum(hh * adst_ref[h:h + 1, :], axis=1)[None, :])
    a1_ref[...] = jnp.concatenate(arow, axis=0)


def _tc_a(x_pad, W1, att_src1, att_dst1):
    grid = (NPAD // BLK_A,)
    return pl.pallas_call(
        _tc_a_body,
        grid=grid,
        in_specs=[
            pl.BlockSpec((BLK_A, D_FEAT), lambda i: (i, 0)),
            pl.BlockSpec((D_FEAT, HEADS * HIDDEN), lambda i: (0, 0)),
            pl.BlockSpec((HEADS, HIDDEN), lambda i: (0, 0)),
            pl.BlockSpec((HEADS, HIDDEN), lambda i: (0, 0)),
        ],
        out_specs=[
            pl.BlockSpec((HEADS, BLK_A, HIDDEN), lambda i: (0, i, 0)),
            pl.BlockSpec((2 * HEADS, BLK_A), lambda i: (0, i)),
        ],
        out_shape=[
            jax.ShapeDtypeStruct((HEADS, NPAD, HIDDEN), jnp.float32),
            jax.ShapeDtypeStruct((2 * HEADS, NPAD), jnp.float32),
        ],
    )(x_pad, W1, att_src1, att_dst1)


def _tc_b_body(p_ref, den_ref, w2_ref, b1_ref, as2_ref, ad2_ref,
               htab_ref, a2_ref):
    p0 = p_ref[0]
    p1 = p_ref[1]
    d0 = jnp.maximum(jnp.sum(den_ref[0], axis=0), 1e-30)[:, None]
    d1 = jnp.maximum(jnp.sum(den_ref[1], axis=0), 1e-30)[:, None]
    h1 = jnp.concatenate([p0 / d0, p1 / d1], axis=1) + b1_ref[...]
    h1 = jnp.where(h1 > 0, h1, jnp.exp(h1) - 1.0)
    hp = jnp.dot(h1, w2_ref[...], preferred_element_type=jnp.float32)
    htab_ref[...] = hp
    a2_ref[...] = jnp.concatenate(
        [jnp.sum(hp * as2_ref[...], axis=1)[None, :],
         jnp.sum(hp * ad2_ref[...], axis=1)[None, :]], axis=0)


def _tc_b(p1v, den1, W2, b1, att_src2, att_dst2):
    grid = (NPAD // BLK_A,)
    return pl.pallas_call(
        _tc_b_body,
        grid=grid,
        in_specs=[
            pl.BlockSpec((2, BLK_A, HIDDEN), lambda i: (0, i, 0)),
            pl.BlockSpec((2, NT, BLK_A), lambda i: (0, 0, i)),
            pl.BlockSpec((HEADS * HIDDEN, HIDDEN), lambda i: (0, 0)),
            pl.BlockSpec((1, HEADS * HIDDEN), lambda i: (0, 0)),
            pl.BlockSpec((1, HIDDEN), lambda i: (0, 0)),
            pl.BlockSpec((1, HIDDEN), lambda i: (0, 0)),
        ],
        out_specs=[
            pl.BlockSpec((BLK_A, HIDDEN), lambda i: (i, 0)),
            pl.BlockSpec((2, BLK_A), lambda i: (0, i)),
        ],
        out_shape=[
            jax.ShapeDtypeStruct((NPAD, HIDDEN), jnp.float32),
            jax.ShapeDtypeStruct((2, NPAD), jnp.float32),
        ],
    )(p1v, den1, W2, b1, att_src2, att_dst2)


def _tc_c_body(p_ref, den_ref, g_ref, be_ref, b2_ref, rf_ref, wr_ref,
               br_ref, out_ref):
    s = p_ref[0] + p_ref[1]
    d = jnp.maximum(jnp.sum(den_ref[0] + den_ref[1], axis=0),
                    1e-30)[:, None]
    h2 = s / d + b2_ref[...]
    mu = jnp.mean(h2, axis=1, keepdims=True)
    var = jnp.mean((h2 - mu) ** 2, axis=1, keepdims=True)
    h2 = (h2 - mu) / jnp.sqrt(var + 1e-5) * g_ref[...] + be_ref[...]
    r = jnp.dot(rf_ref[...], wr_ref[...],
                preferred_element_type=jnp.float32) + br_ref[...]
    r = jnp.where(r > 0, r, jnp.exp(r) - 1.0)
    out_ref[:, :, :HIDDEN] = jnp.broadcast_to(
        h2[:, None, :], (BLK_C, N_RESOURCES, HIDDEN))
    out_ref[:, :, HIDDEN:] = jnp.broadcast_to(
        r[None, :, :], (BLK_C, N_RESOURCES, HIDDEN))


def _tc_c(p2v, den2, gamma, beta, b2, rf, Wr, br):
    grid = (NPAD // BLK_C,)
    return pl.pallas_call(
        _tc_c_body,
        grid=grid,
        in_specs=[
            pl.BlockSpec((2, BLK_C, HIDDEN), lambda i: (0, i, 0)),
            pl.BlockSpec((2, NT, BLK_C), lambda i: (0, 0, i)),
            pl.BlockSpec((1, HIDDEN), lambda i: (0, 0)),
            pl.BlockSpec((1, HIDDEN), lambda i: (0, 0)),
            pl.BlockSpec((1, HIDDEN), lambda i: (0, 0)),
            pl.BlockSpec((N_RESOURCES, 32), lambda i: (0, 0)),
            pl.BlockSpec((32, HIDDEN), lambda i: (0, 0)),
            pl.BlockSpec((1, HIDDEN), lambda i: (0, 0)),
        ],
        out_specs=pl.BlockSpec((BLK_C, N_RESOURCES, 2 * HIDDEN),
                               lambda i: (i, 0, 0)),
        out_shape=jax.ShapeDtypeStruct((N_NODES, N_RESOURCES, 2 * HIDDEN),
                                       jnp.float32),
    )(p2v, den2, gamma, beta, b2, rf, Wr, br)


def _make_sc_pass(two_tables, chunks_per_tile):
    mesh = plsc.VectorSubcoreMesh(core_axis_name="c", subcore_axis_name="s")

    @functools.partial(
        pl.kernel, mesh=mesh,
        compiler_params=pltpu.CompilerParams(needs_layout_passes=False),
        out_type=[
            jax.ShapeDtypeStruct((2 * NPAD, HIDDEN), jnp.float32),
            jax.ShapeDtypeStruct((2 * NT * NPAD,), jnp.float32),
        ],
        scratch_types=[
            pltpu.VMEM((NPAD,), jnp.float32),
            pltpu.VMEM((NPAD,), jnp.float32),
            pltpu.VMEM((NPAD,), jnp.float32),
            pltpu.VMEM((K,), jnp.int32),
            pltpu.VMEM((K,), jnp.int32),
            pltpu.VMEM((K,), jnp.int32),
            pltpu.VMEM((K,), jnp.int32),
            pltpu.VMEM((K,), jnp.int32),
            pltpu.VMEM((K,), jnp.float32),
            pltpu.VMEM((K, HIDDEN), jnp.float32),
            pltpu.VMEM((K, HIDDEN), jnp.float32),
            pltpu.VMEM_SHARED((NPAD, HIDDEN), jnp.float32),
            pltpu.SemaphoreType.DMA,
            pltpu.SemaphoreType.DMA,
            pltpu.SemaphoreType.DMA,
            pltpu.SemaphoreType.DMA,
            pltpu.SemaphoreType.DMA,
            pltpu.SemaphoreType.DMA,
            pltpu.SemaphoreType.DMA,
        ])
    def sc_pass(htab, aflat, src_hbm, dst_hbm, zeros_hbm, zeros1_hbm,
                out_hbm, den_hbm,
                asrc_v, adst_v, denom_v, src0, src1, dst0, dst1, gidx_v,
                w_buf, rows0, rows1, out_sh, sem_rows,
                sisrc0, sisrc1, sidst0, sidst1, sscat0, sscat1):
        src_ring = [src0, src1]
        dst_ring = [dst0, dst1]
        rows_ring = [rows0, rows1]
        sem_isrc = [sisrc0, sisrc1]
        sem_idst = [sidst0, sidst1]
        sem_scat = [sscat0, sscat1]
        c = lax.axis_index("c")
        s = lax.axis_index("s")
        pltpu.sync_copy(zeros_hbm.at[pl.ds(s * RPT, RPT)],
                        out_sh.at[pl.ds(s * RPT, RPT)])
        pltpu.sync_copy(zeros1_hbm, denom_v)
        if two_tables:
            a_base = (2 * c) * NPAD
            tab_off = c * NPAD
            tile_base = s * chunks_per_tile * K
        else:
            a_base = 0
            tab_off = 0
            tile_base = (s * 2 + c) * chunks_per_tile * K
        pltpu.sync_copy(aflat.at[pl.ds(a_base, NPAD)], asrc_v)
        pltpu.sync_copy(aflat.at[pl.ds(a_base + NPAD, NPAD)], adst_v)
        plsc.subcore_barrier()

        C = chunks_per_tile

        def issue_idx(i, b):
            base = tile_base + i * K
            pltpu.async_copy(src_hbm.at[pl.ds(base, K)], src_ring[b],
                             sem_isrc[b])
            pltpu.async_copy(dst_hbm.at[pl.ds(base, K)], dst_ring[b],
                             sem_idst[b])

        def wait_idx(i, b):
            base = tile_base + i * K
            pltpu.make_async_copy(src_hbm.at[pl.ds(base, K)], src_ring[b],
                                  sem_isrc[b]).wait()
            pltpu.make_async_copy(dst_hbm.at[pl.ds(base, K)], dst_ring[b],
                                  sem_idst[b]).wait()

        issue_idx(0, 0)

        def do_chunk(i, b):
            o = 1 - b
            src_v = src_ring[b]
            dst_v = dst_ring[b]
            rows_v = rows_ring[b]
            wait_idx(i, b)
            for j in range(K // 16):
                sl = pl.ds(j * 16, 16)
                gidx_v[sl] = src_v[sl] + tab_off
            gather = pltpu.async_copy(htab.at[gidx_v], rows_v, sem_rows)
            for j in range(K // 16):
                sl = pl.ds(j * 16, 16)
                si = src_v[sl]
                di = dst_v[sl]
                av = plsc.load_gather(asrc_v, [si])
                dv = plsc.load_gather(adst_v, [di])
                al = av + dv
                al = jnp.maximum(al, 0.0) + 0.2 * jnp.minimum(al, 0.0)
                w = jnp.exp(al)
                w_buf[sl] = w
                plsc.addupdate_scatter(denom_v, [di], w)
            @pl.when(i >= 1)
            def _():
                pltpu.make_async_copy(rows_ring[o], out_sh.at[dst_ring[o]],
                                      sem_scat[o]).wait()

            @pl.when(i + 1 < C)
            def _():
                issue_idx(i + 1, o)

            gather.wait()
            for g in range(K // 16):
                wg = w_buf[pl.ds(g * 16, 16)]
                for e16 in range(16):
                    wv = jnp.full((16,), wg[e16], jnp.float32)
                    e = g * 16 + e16
                    for j in range(HIDDEN // 16):
                        sl2 = pl.ds(j * 16, 16)
                        rows_v[e, sl2] = rows_v[e, sl2] * wv
            pltpu.async_copy(rows_v, out_sh.at[dst_v], sem_scat[b],
                             add=True)

        def round_(r, carry):
            do_chunk(2 * r, 0)
            do_chunk(2 * r + 1, 1)
            return carry

        lax.fori_loop(0, C // 2, round_, 0)
        pltpu.make_async_copy(rows_ring[1], out_sh.at[dst_ring[1]],
                              sem_scat[1]).wait()
        pltpu.sync_copy(denom_v,
                        den_hbm.at[pl.ds((c * NT + s) * NPAD, NPAD)])
        plsc.subcore_barrier()
        pltpu.sync_copy(out_sh.at[pl.ds(s * RPT, RPT)],
                        out_hbm.at[pl.ds(c * NPAD + s * RPT, RPT)])

    return sc_pass


_make_sc_pass = functools.lru_cache(maxsize=None)(_make_sc_pass)


def kernel(x, edge_index, batch, resource_features,
           W1, att_src1, att_dst1, b1,
           W2, att_src2, att_dst2, b2,
           Wr, br, gamma, beta):
    n = x.shape[0]
    loops = jnp.arange(n, dtype=jnp.int32)
    fill = jnp.full((EPAD - E_LOOPS,), n, jnp.int32)
    src = jnp.concatenate([edge_index[0].astype(jnp.int32), loops, fill])
    dst = jnp.concatenate([edge_index[1].astype(jnp.int32), loops, fill])

    x_pad = jnp.zeros((NPAD, D_FEAT), jnp.float32).at[:n].set(x)
    zeros_tab = jnp.zeros((NPAD, HIDDEN), jnp.float32)
    zeros_vec = jnp.zeros((NPAD,), jnp.float32)

    _sc_layer1 = _make_sc_pass(True, CH1)
    _sc_layer2 = _make_sc_pass(False, CH2)

    htab1, a1 = _tc_a(x_pad, W1, att_src1, att_dst1)
    out1, den1 = _sc_layer1(htab1.reshape(HEADS * NPAD, HIDDEN),
                            a1.reshape(2 * HEADS * NPAD), src, dst,
                            zeros_tab, zeros_vec)
    htab2, a2 = _tc_b(out1.reshape(2, NPAD, HIDDEN),
                      den1.reshape(2, NT, NPAD), W2,
                      b1.reshape(1, HEADS * HIDDEN),
                      att_src2, att_dst2)
    out2, den2 = _sc_layer2(htab2, a2.reshape(2 * NPAD), src, dst,
                            zeros_tab, zeros_vec)
    return _tc_c(out2.reshape(2, NPAD, HIDDEN), den2.reshape(2, NT, NPAD),
                 gamma.reshape(1, HIDDEN), beta.reshape(1, HIDDEN),
                 b2.reshape(1, HIDDEN), resource_features, Wr,
                 br.reshape(1, HIDDEN))


# speedup vs baseline: 47.4259x; 1.1687x over previous
import functools

import jax
import jax.numpy as jnp
from jax import lax
from jax.experimental import pallas as pl
from jax.experimental.pallas import tpu as pltpu
from jax.experimental.pallas import tpu_sc as plsc

N_NODES = 10000
D_FEAT = 128
HIDDEN = 128
HEADS = 2
N_RESOURCES = 16

NPAD = 10240
NA = 10048
K = 32
E_RAW = 320000
E_LOOPS = E_RAW + N_NODES
CH1 = 645
CH2 = 324
EPAD = 331776
NT = 16
RPT = NPAD // NT

BLK_A = 640
BLK_C = 256


def _tc_a_body(x_ref, w1_ref, asrc_ref, adst_ref, htab_ref, a1_ref):
    xb = x_ref[...]
    hb = jnp.dot(xb, w1_ref[...], preferred_element_type=jnp.float32)
    arow = []
    for h in range(HEADS):
        hh = hb[:, h * HIDDEN:(h + 1) * HIDDEN]
        htab_ref[h] = hh
        arow.append(jnp.sum(hh * asrc_ref[h:h + 1, :], axis=1)[None, :])
        arow.append(jnp.sum(hh * adst_ref[h:h + 1, :], axis=1)[None, :])
    a1_ref[...] = jnp.concatenate(arow, axis=0)


def _tc_a(x_pad, W1, att_src1, att_dst1):
    grid = (NPAD // BLK_A,)
    return pl.pallas_call(
        _tc_a_body,
        grid=grid,
        in_specs=[
            pl.BlockSpec((BLK_A, D_FEAT), lambda i: (i, 0)),
            pl.BlockSpec((D_FEAT, HEADS * HIDDEN), lambda i: (0, 0)),
            pl.BlockSpec((HEADS, HIDDEN), lambda i: (0, 0)),
            pl.BlockSpec((HEADS, HIDDEN), lambda i: (0, 0)),
        ],
        out_specs=[
            pl.BlockSpec((HEADS, BLK_A, HIDDEN), lambda i: (0, i, 0)),
            pl.BlockSpec((2 * HEADS, BLK_A), lambda i: (0, i)),
        ],
        out_shape=[
            jax.ShapeDtypeStruct((HEADS, NPAD, HIDDEN), jnp.float32),
            jax.ShapeDtypeStruct((2 * HEADS, NPAD), jnp.float32),
        ],
    )(x_pad, W1, att_src1, att_dst1)


def _tc_b_body(p_ref, den_ref, w2_ref, b1_ref, as2_ref, ad2_ref,
               htab_ref, a2_ref):
    p0 = p_ref[0]
    p1 = p_ref[1]
    d0 = jnp.maximum(jnp.sum(den_ref[0], axis=0), 1e-30)[:, None]
    d1 = jnp.maximum(jnp.sum(den_ref[1], axis=0), 1e-30)[:, None]
    h1 = jnp.concatenate([p0 / d0, p1 / d1], axis=1) + b1_ref[...]
    h1 = jnp.where(h1 > 0, h1, jnp.exp(h1) - 1.0)
    hp = jnp.dot(h1, w2_ref[...], preferred_element_type=jnp.float32)
    htab_ref[...] = hp
    a2_ref[...] = jnp.concatenate(
        [jnp.sum(hp * as2_ref[...], axis=1)[None, :],
         jnp.sum(hp * ad2_ref[...], axis=1)[None, :]], axis=0)


def _tc_b(p1v, den1, W2, b1, att_src2, att_dst2):
    grid = (NPAD // BLK_A,)
    return pl.pallas_call(
        _tc_b_body,
        grid=grid,
        in_specs=[
            pl.BlockSpec((2, BLK_A, HIDDEN), lambda i: (0, i, 0)),
            pl.BlockSpec((2, NT, BLK_A), lambda i: (0, 0, i)),
            pl.BlockSpec((HEADS * HIDDEN, HIDDEN), lambda i: (0, 0)),
            pl.BlockSpec((1, HEADS * HIDDEN), lambda i: (0, 0)),
            pl.BlockSpec((1, HIDDEN), lambda i: (0, 0)),
            pl.BlockSpec((1, HIDDEN), lambda i: (0, 0)),
        ],
        out_specs=[
            pl.BlockSpec((BLK_A, HIDDEN), lambda i: (i, 0)),
            pl.BlockSpec((2, BLK_A), lambda i: (0, i)),
        ],
        out_shape=[
            jax.ShapeDtypeStruct((NPAD, HIDDEN), jnp.float32),
            jax.ShapeDtypeStruct((2, NPAD), jnp.float32),
        ],
    )(p1v, den1, W2, b1, att_src2, att_dst2)


def _tc_c_body(p_ref, den_ref, g_ref, be_ref, b2_ref, rf_ref, wr_ref,
               br_ref, out_ref):
    s = p_ref[0] + p_ref[1]
    d = jnp.maximum(jnp.sum(den_ref[0] + den_ref[1], axis=0),
                    1e-30)[:, None]
    h2 = s / d + b2_ref[...]
    mu = jnp.mean(h2, axis=1, keepdims=True)
    var = jnp.mean((h2 - mu) ** 2, axis=1, keepdims=True)
    h2 = (h2 - mu) / jnp.sqrt(var + 1e-5) * g_ref[...] + be_ref[...]
    r = jnp.dot(rf_ref[...], wr_ref[...],
                preferred_element_type=jnp.float32) + br_ref[...]
    r = jnp.where(r > 0, r, jnp.exp(r) - 1.0)
    out_ref[:, :, :HIDDEN] = jnp.broadcast_to(
        h2[:, None, :], (BLK_C, N_RESOURCES, HIDDEN))
    out_ref[:, :, HIDDEN:] = jnp.broadcast_to(
        r[None, :, :], (BLK_C, N_RESOURCES, HIDDEN))


def _tc_c(p2v, den2, gamma, beta, b2, rf, Wr, br):
    grid = (NPAD // BLK_C,)
    return pl.pallas_call(
        _tc_c_body,
        grid=grid,
        in_specs=[
            pl.BlockSpec((2, BLK_C, HIDDEN), lambda i: (0, i, 0)),
            pl.BlockSpec((2, NT, BLK_C), lambda i: (0, 0, i)),
            pl.BlockSpec((1, HIDDEN), lambda i: (0, 0)),
            pl.BlockSpec((1, HIDDEN), lambda i: (0, 0)),
            pl.BlockSpec((1, HIDDEN), lambda i: (0, 0)),
            pl.BlockSpec((N_RESOURCES, 32), lambda i: (0, 0)),
            pl.BlockSpec((32, HIDDEN), lambda i: (0, 0)),
            pl.BlockSpec((1, HIDDEN), lambda i: (0, 0)),
        ],
        out_specs=pl.BlockSpec((BLK_C, N_RESOURCES, 2 * HIDDEN),
                               lambda i: (i, 0, 0)),
        out_shape=jax.ShapeDtypeStruct((N_NODES, N_RESOURCES, 2 * HIDDEN),
                                       jnp.float32),
    )(p2v, den2, gamma, beta, b2, rf, Wr, br)


def _make_sc_pass(two_tables, chunks_per_tile):
    mesh = plsc.VectorSubcoreMesh(core_axis_name="c", subcore_axis_name="s")

    @functools.partial(
        pl.kernel, mesh=mesh,
        compiler_params=pltpu.CompilerParams(needs_layout_passes=False),
        out_type=[
            jax.ShapeDtypeStruct((2 * NPAD, HIDDEN), jnp.float32),
            jax.ShapeDtypeStruct((2 * NT * NPAD,), jnp.float32),
        ],
        scratch_types=[
            pltpu.VMEM((NA,), jnp.float32),
            pltpu.VMEM((NA,), jnp.float32),
            pltpu.VMEM((NA,), jnp.float32),
            pltpu.VMEM((K,), jnp.int32),
            pltpu.VMEM((K,), jnp.int32),
            pltpu.VMEM((K,), jnp.int32),
            pltpu.VMEM((K,), jnp.int32),
            pltpu.VMEM((K,), jnp.int32),
            pltpu.VMEM((K,), jnp.int32),
            pltpu.VMEM((K,), jnp.int32),
            pltpu.VMEM((K,), jnp.int32),
            pltpu.VMEM((K,), jnp.int32),
            pltpu.VMEM((K,), jnp.float32),
            pltpu.VMEM((K, HIDDEN), jnp.float32),
            pltpu.VMEM((K, HIDDEN), jnp.float32),
            pltpu.VMEM((K, HIDDEN), jnp.float32),
            pltpu.VMEM_SHARED((NPAD, HIDDEN), jnp.float32),
            pltpu.SemaphoreType.DMA,
            pltpu.SemaphoreType.DMA,
            pltpu.SemaphoreType.DMA,
            pltpu.SemaphoreType.DMA,
            pltpu.SemaphoreType.DMA,
            pltpu.SemaphoreType.DMA,
            pltpu.SemaphoreType.DMA,
            pltpu.SemaphoreType.DMA,
            pltpu.SemaphoreType.DMA,
            pltpu.SemaphoreType.DMA,
            pltpu.SemaphoreType.DMA,
            pltpu.SemaphoreType.DMA,
        ])
    def sc_pass(htab, aflat, src_hbm, dst_hbm, zeros_hbm, zeros1_hbm,
                out_hbm, den_hbm,
                asrc_v, adst_v, denom_v, src0, src1, src2, dst0, dst1,
                dst2, gi0, gi1, gi2, w_buf, rows0, rows1, rows2, out_sh,
                sr0, sr1, sr2, sisrc0, sisrc1, sisrc2,
                sidst0, sidst1, sidst2, sscat0, sscat1, sscat2):
        src_ring = [src0, src1, src2]
        dst_ring = [dst0, dst1, dst2]
        gidx_ring = [gi0, gi1, gi2]
        rows_ring = [rows0, rows1, rows2]
        sem_rows = [sr0, sr1, sr2]
        sem_isrc = [sisrc0, sisrc1, sisrc2]
        sem_idst = [sidst0, sidst1, sidst2]
        sem_scat = [sscat0, sscat1, sscat2]
        c = lax.axis_index("c")
        s = lax.axis_index("s")
        pltpu.sync_copy(zeros_hbm.at[pl.ds(s * RPT, RPT)],
                        out_sh.at[pl.ds(s * RPT, RPT)])
        pltpu.sync_copy(zeros1_hbm.at[pl.ds(0, NA)], denom_v)
        if two_tables:
            a_base = (2 * c) * NPAD
            tab_off = c * NPAD
            tile_base = s * chunks_per_tile * K
        else:
            a_base = 0
            tab_off = 0
            tile_base = (s * 2 + c) * chunks_per_tile * K
        pltpu.sync_copy(aflat.at[pl.ds(a_base, NA)], asrc_v)
        pltpu.sync_copy(aflat.at[pl.ds(a_base + NPAD, NA)], adst_v)
        plsc.subcore_barrier()

        C = chunks_per_tile

        def issue_idx(i, b):
            base = tile_base + i * K
            pltpu.async_copy(src_hbm.at[pl.ds(base, K)], src_ring[b],
                             sem_isrc[b])
            pltpu.async_copy(dst_hbm.at[pl.ds(base, K)], dst_ring[b],
                             sem_idst[b])

        def wait_idx(i, b):
            base = tile_base + i * K
            pltpu.make_async_copy(src_hbm.at[pl.ds(base, K)], src_ring[b],
                                  sem_isrc[b]).wait()
            pltpu.make_async_copy(dst_hbm.at[pl.ds(base, K)], dst_ring[b],
                                  sem_idst[b]).wait()

        def issue_gather(i, b):
            for j in range(K // 16):
                sl = pl.ds(j * 16, 16)
                gidx_ring[b][sl] = src_ring[b][sl] + tab_off
            pltpu.async_copy(htab.at[gidx_ring[b]], rows_ring[b],
                             sem_rows[b])

        issue_idx(0, 0)
        issue_idx(1, 1)
        wait_idx(0, 0)
        issue_gather(0, 0)

        def do_chunk(i, b):
            nxt = (b + 1) % 3
            ovr = (b + 2) % 3
            src_v = src_ring[b]
            dst_v = dst_ring[b]
            rows_v = rows_ring[b]
            for j in range(K // 16):
                sl = pl.ds(j * 16, 16)
                si = src_v[sl]
                di = dst_v[sl]
                av = plsc.load_gather(asrc_v, [si])
                dv = plsc.load_gather(adst_v, [di])
                al = av + dv
                al = jnp.maximum(al, 0.0) + 0.2 * jnp.minimum(al, 0.0)
                w = jnp.exp(al)
                w_buf[sl] = w
                plsc.addupdate_scatter(denom_v, [di], w)
            @pl.when(i >= 1)
            def _():
                pltpu.make_async_copy(rows_ring[ovr],
                                      out_sh.at[dst_ring[ovr]],
                                      sem_scat[ovr]).wait()

            @pl.when(i + 2 < C)
            def _():
                issue_idx(i + 2, ovr)

            @pl.when(i + 1 < C)
            def _():
                wait_idx(i + 1, nxt)
                issue_gather(i + 1, nxt)

            pltpu.make_async_copy(htab.at[gidx_ring[b]], rows_v,
                                  sem_rows[b]).wait()
            for g in range(K // 16):
                wg = w_buf[pl.ds(g * 16, 16)]
                for e16 in range(16):
                    wv = jnp.full((16,), wg[e16], jnp.float32)
                    e = g * 16 + e16
                    for j in range(HIDDEN // 16):
                        sl2 = pl.ds(j * 16, 16)
                        rows_v[e, sl2] = rows_v[e, sl2] * wv
            pltpu.async_copy(rows_v, out_sh.at[dst_v], sem_scat[b],
                             add=True)

        def round_(r, carry):
            do_chunk(3 * r, 0)
            do_chunk(3 * r + 1, 1)
            do_chunk(3 * r + 2, 2)
            return carry

        lax.fori_loop(0, C // 3, round_, 0)
        fb = (C - 1) % 3
        pltpu.make_async_copy(rows_ring[fb], out_sh.at[dst_ring[fb]],
                              sem_scat[fb]).wait()
        pltpu.sync_copy(denom_v,
                        den_hbm.at[pl.ds((c * NT + s) * NPAD, NA)])
        plsc.subcore_barrier()
        pltpu.sync_copy(out_sh.at[pl.ds(s * RPT, RPT)],
                        out_hbm.at[pl.ds(c * NPAD + s * RPT, RPT)])

    return sc_pass


_make_sc_pass = functools.lru_cache(maxsize=None)(_make_sc_pass)


def kernel(x, edge_index, batch, resource_features,
           W1, att_src1, att_dst1, b1,
           W2, att_src2, att_dst2, b2,
           Wr, br, gamma, beta):
    n = x.shape[0]
    loops = jnp.arange(n, dtype=jnp.int32)
    fill = jnp.full((EPAD - E_LOOPS,), n, jnp.int32)
    src = jnp.concatenate([edge_index[0].astype(jnp.int32), loops, fill])
    dst = jnp.concatenate([edge_index[1].astype(jnp.int32), loops, fill])

    x_pad = jnp.zeros((NPAD, D_FEAT), jnp.float32).at[:n].set(x)
    zeros_tab = jnp.zeros((NPAD, HIDDEN), jnp.float32)
    zeros_vec = jnp.zeros((NPAD,), jnp.float32)

    _sc_layer1 = _make_sc_pass(True, CH1)
    _sc_layer2 = _make_sc_pass(False, CH2)

    htab1, a1 = _tc_a(x_pad, W1, att_src1, att_dst1)
    out1, den1 = _sc_layer1(htab1.reshape(HEADS * NPAD, HIDDEN),
                            a1.reshape(2 * HEADS * NPAD), src, dst,
                            zeros_tab, zeros_vec)
    htab2, a2 = _tc_b(out1.reshape(2, NPAD, HIDDEN),
                      den1.reshape(2, NT, NPAD), W2,
                      b1.reshape(1, HEADS * HIDDEN),
                      att_src2, att_dst2)
    out2, den2 = _sc_layer2(htab2, a2.reshape(2 * NPAD), src, dst,
                            zeros_tab, zeros_vec)
    return _tc_c(out2.reshape(2, NPAD, HIDDEN), den2.reshape(2, NT, NPAD),
                 gamma.reshape(1, HIDDEN), beta.reshape(1, HIDDEN),
                 b2.reshape(1, HIDDEN), resource_features, Wr,
                 br.reshape(1, HIDDEN))


# speedup vs baseline: 49.6504x; 1.0469x over previous
import functools

import jax
import jax.numpy as jnp
from jax import lax
from jax.experimental import pallas as pl
from jax.experimental.pallas import tpu as pltpu
from jax.experimental.pallas import tpu_sc as plsc

N_NODES = 10000
D_FEAT = 128
HIDDEN = 128
HEADS = 2
N_RESOURCES = 16

NPAD = 10240
NA = 10048
K = 48
E_RAW = 320000
E_LOOPS = E_RAW + N_NODES
CH1 = 432
CH2 = 216
EPAD = 331776
NT = 16
RPT = NPAD // NT
NACC = 10112
RA = NACC // NT

BLK_A = 640
BLK_C = 256


def _tc_a_body(x_ref, w1_ref, asrc_ref, adst_ref, htab_ref, a1_ref):
    xb = x_ref[...]
    hb = jnp.dot(xb, w1_ref[...], preferred_element_type=jnp.float32)
    arow = []
    for h in range(HEADS):
        hh = hb[:, h * HIDDEN:(h + 1) * HIDDEN]
        htab_ref[h] = hh
        arow.append(jnp.sum(hh * asrc_ref[h:h + 1, :], axis=1)[None, :])
        arow.append(jnp.sum(hh * adst_ref[h:h + 1, :], axis=1)[None, :])
    a1_ref[...] = jnp.concatenate(arow, axis=0)


def _tc_a(x_pad, W1, att_src1, att_dst1):
    grid = (NPAD // BLK_A,)
    return pl.pallas_call(
        _tc_a_body,
        grid=grid,
        in_specs=[
            pl.BlockSpec((BLK_A, D_FEAT), lambda i: (i, 0)),
            pl.BlockSpec((D_FEAT, HEADS * HIDDEN), lambda i: (0, 0)),
            pl.BlockSpec((HEADS, HIDDEN), lambda i: (0, 0)),
            pl.BlockSpec((HEADS, HIDDEN), lambda i: (0, 0)),
        ],
        out_specs=[
            pl.BlockSpec((HEADS, BLK_A, HIDDEN), lambda i: (0, i, 0)),
            pl.BlockSpec((2 * HEADS, BLK_A), lambda i: (0, i)),
        ],
        out_shape=[
            jax.ShapeDtypeStruct((HEADS, NPAD, HIDDEN), jnp.float32),
            jax.ShapeDtypeStruct((2 * HEADS, NPAD), jnp.float32),
        ],
    )(x_pad, W1, att_src1, att_dst1)


def _tc_b_body(p_ref, den_ref, w2_ref, b1_ref, as2_ref, ad2_ref,
               htab_ref, a2_ref):
    p0 = p_ref[0]
    p1 = p_ref[1]
    d0 = jnp.maximum(jnp.sum(den_ref[0], axis=0), 1e-30)[:, None]
    d1 = jnp.maximum(jnp.sum(den_ref[1], axis=0), 1e-30)[:, None]
    h1 = jnp.concatenate([p0 / d0, p1 / d1], axis=1) + b1_ref[...]
    h1 = jnp.where(h1 > 0, h1, jnp.exp(h1) - 1.0)
    hp = jnp.dot(h1, w2_ref[...], preferred_element_type=jnp.float32)
    htab_ref[...] = hp
    a2_ref[...] = jnp.concatenate(
        [jnp.sum(hp * as2_ref[...], axis=1)[None, :],
         jnp.sum(hp * ad2_ref[...], axis=1)[None, :]], axis=0)


def _tc_b(p1v, den1, W2, b1, att_src2, att_dst2):
    grid = (NPAD // BLK_A,)
    return pl.pallas_call(
        _tc_b_body,
        grid=grid,
        in_specs=[
            pl.BlockSpec((2, BLK_A, HIDDEN), lambda i: (0, i, 0)),
            pl.BlockSpec((2, NT, BLK_A), lambda i: (0, 0, i)),
            pl.BlockSpec((HEADS * HIDDEN, HIDDEN), lambda i: (0, 0)),
            pl.BlockSpec((1, HEADS * HIDDEN), lambda i: (0, 0)),
            pl.BlockSpec((1, HIDDEN), lambda i: (0, 0)),
            pl.BlockSpec((1, HIDDEN), lambda i: (0, 0)),
        ],
        out_specs=[
            pl.BlockSpec((BLK_A, HIDDEN), lambda i: (i, 0)),
            pl.BlockSpec((2, BLK_A), lambda i: (0, i)),
        ],
        out_shape=[
            jax.ShapeDtypeStruct((NPAD, HIDDEN), jnp.float32),
            jax.ShapeDtypeStruct((2, NPAD), jnp.float32),
        ],
    )(p1v, den1, W2, b1, att_src2, att_dst2)


def _tc_c_body(p_ref, den_ref, g_ref, be_ref, b2_ref, rf_ref, wr_ref,
               br_ref, out_ref):
    s = p_ref[0] + p_ref[1]
    d = jnp.maximum(jnp.sum(den_ref[0] + den_ref[1], axis=0),
                    1e-30)[:, None]
    h2 = s / d + b2_ref[...]
    mu = jnp.mean(h2, axis=1, keepdims=True)
    var = jnp.mean((h2 - mu) ** 2, axis=1, keepdims=True)
    h2 = (h2 - mu) / jnp.sqrt(var + 1e-5) * g_ref[...] + be_ref[...]
    r = jnp.dot(rf_ref[...], wr_ref[...],
                preferred_element_type=jnp.float32) + br_ref[...]
    r = jnp.where(r > 0, r, jnp.exp(r) - 1.0)
    out_ref[:, :, :HIDDEN] = jnp.broadcast_to(
        h2[:, None, :], (BLK_C, N_RESOURCES, HIDDEN))
    out_ref[:, :, HIDDEN:] = jnp.broadcast_to(
        r[None, :, :], (BLK_C, N_RESOURCES, HIDDEN))


def _tc_c(p2v, den2, gamma, beta, b2, rf, Wr, br):
    grid = (NPAD // BLK_C,)
    return pl.pallas_call(
        _tc_c_body,
        grid=grid,
        in_specs=[
            pl.BlockSpec((2, BLK_C, HIDDEN), lambda i: (0, i, 0)),
            pl.BlockSpec((2, NT, BLK_C), lambda i: (0, 0, i)),
            pl.BlockSpec((1, HIDDEN), lambda i: (0, 0)),
            pl.BlockSpec((1, HIDDEN), lambda i: (0, 0)),
            pl.BlockSpec((1, HIDDEN), lambda i: (0, 0)),
            pl.BlockSpec((N_RESOURCES, 32), lambda i: (0, 0)),
            pl.BlockSpec((32, HIDDEN), lambda i: (0, 0)),
            pl.BlockSpec((1, HIDDEN), lambda i: (0, 0)),
        ],
        out_specs=pl.BlockSpec((BLK_C, N_RESOURCES, 2 * HIDDEN),
                               lambda i: (i, 0, 0)),
        out_shape=jax.ShapeDtypeStruct((N_NODES, N_RESOURCES, 2 * HIDDEN),
                                       jnp.float32),
    )(p2v, den2, gamma, beta, b2, rf, Wr, br)


def _make_sc_pass(two_tables, chunks_per_tile):
    mesh = plsc.VectorSubcoreMesh(core_axis_name="c", subcore_axis_name="s")

    @functools.partial(
        pl.kernel, mesh=mesh,
        compiler_params=pltpu.CompilerParams(needs_layout_passes=False),
        out_type=[
            jax.ShapeDtypeStruct((2 * NPAD, HIDDEN), jnp.float32),
            jax.ShapeDtypeStruct((2 * NT * NPAD,), jnp.float32),
        ],
        scratch_types=[
            pltpu.VMEM((NA,), jnp.float32),
            pltpu.VMEM((NA,), jnp.float32),
            pltpu.VMEM((NA,), jnp.float32),
            pltpu.VMEM((K,), jnp.int32),
            pltpu.VMEM((K,), jnp.int32),
            pltpu.VMEM((K,), jnp.int32),
            pltpu.VMEM((K,), jnp.int32),
            pltpu.VMEM((K,), jnp.int32),
            pltpu.VMEM((K,), jnp.int32),
            pltpu.VMEM((K,), jnp.int32),
            pltpu.VMEM((K,), jnp.int32),
            pltpu.VMEM((K,), jnp.int32),
            pltpu.VMEM((K,), jnp.float32),
            pltpu.VMEM((K, HIDDEN), jnp.float32),
            pltpu.VMEM((K, HIDDEN), jnp.float32),
            pltpu.VMEM((K, HIDDEN), jnp.float32),
            pltpu.VMEM_SHARED((NACC, HIDDEN), jnp.float32),
            pltpu.SemaphoreType.DMA,
            pltpu.SemaphoreType.DMA,
            pltpu.SemaphoreType.DMA,
            pltpu.SemaphoreType.DMA,
            pltpu.SemaphoreType.DMA,
            pltpu.SemaphoreType.DMA,
            pltpu.SemaphoreType.DMA,
            pltpu.SemaphoreType.DMA,
            pltpu.SemaphoreType.DMA,
            pltpu.SemaphoreType.DMA,
            pltpu.SemaphoreType.DMA,
            pltpu.SemaphoreType.DMA,
        ])
    def sc_pass(htab, aflat, src_hbm, dst_hbm, zeros_hbm, zeros1_hbm,
                out_hbm, den_hbm,
                asrc_v, adst_v, denom_v, src0, src1, src2, dst0, dst1,
                dst2, gi0, gi1, gi2, w_buf, rows0, rows1, rows2, out_sh,
                sr0, sr1, sr2, sisrc0, sisrc1, sisrc2,
                sidst0, sidst1, sidst2, sscat0, sscat1, sscat2):
        src_ring = [src0, src1, src2]
        dst_ring = [dst0, dst1, dst2]
        gidx_ring = [gi0, gi1, gi2]
        rows_ring = [rows0, rows1, rows2]
        sem_rows = [sr0, sr1, sr2]
        sem_isrc = [sisrc0, sisrc1, sisrc2]
        sem_idst = [sidst0, sidst1, sidst2]
        sem_scat = [sscat0, sscat1, sscat2]
        c = lax.axis_index("c")
        s = lax.axis_index("s")
        pltpu.sync_copy(zeros_hbm.at[pl.ds(s * RA, RA)],
                        out_sh.at[pl.ds(s * RA, RA)])
        pltpu.sync_copy(zeros1_hbm.at[pl.ds(0, NA)], denom_v)
        if two_tables:
            a_base = (2 * c) * NPAD
            tab_off = c * NPAD
            tile_base = s * chunks_per_tile * K
        else:
            a_base = 0
            tab_off = 0
            tile_base = (s * 2 + c) * chunks_per_tile * K
        pltpu.sync_copy(aflat.at[pl.ds(a_base, NA)], asrc_v)
        pltpu.sync_copy(aflat.at[pl.ds(a_base + NPAD, NA)], adst_v)
        plsc.subcore_barrier()

        C = chunks_per_tile

        def issue_idx(i, b):
            base = tile_base + i * K
            pltpu.async_copy(src_hbm.at[pl.ds(base, K)], src_ring[b],
                             sem_isrc[b])
            pltpu.async_copy(dst_hbm.at[pl.ds(base, K)], dst_ring[b],
                             sem_idst[b])

        def wait_idx(i, b):
            base = tile_base + i * K
            pltpu.make_async_copy(src_hbm.at[pl.ds(base, K)], src_ring[b],
                                  sem_isrc[b]).wait()
            pltpu.make_async_copy(dst_hbm.at[pl.ds(base, K)], dst_ring[b],
                                  sem_idst[b]).wait()

        def issue_gather(i, b):
            for j in range(K // 16):
                sl = pl.ds(j * 16, 16)
                gidx_ring[b][sl] = src_ring[b][sl] + tab_off
            pltpu.async_copy(htab.at[gidx_ring[b]], rows_ring[b],
                             sem_rows[b])

        issue_idx(0, 0)
        issue_idx(1, 1)
        wait_idx(0, 0)
        issue_gather(0, 0)

        def do_chunk(i, b):
            nxt = (b + 1) % 3
            ovr = (b + 2) % 3
            src_v = src_ring[b]
            dst_v = dst_ring[b]
            rows_v = rows_ring[b]
            for j in range(K // 16):
                sl = pl.ds(j * 16, 16)
                si = src_v[sl]
                di = dst_v[sl]
                av = plsc.load_gather(asrc_v, [si])
                dv = plsc.load_gather(adst_v, [di])
                al = av + dv
                al = jnp.maximum(al, 0.0) + 0.2 * jnp.minimum(al, 0.0)
                w = jnp.exp(al)
                w_buf[sl] = w
                plsc.addupdate_scatter(denom_v, [di], w)
            @pl.when(i >= 1)
            def _():
                pltpu.make_async_copy(rows_ring[ovr],
                                      out_sh.at[dst_ring[ovr]],
                                      sem_scat[ovr]).wait()

            @pl.when(i + 2 < C)
            def _():
                issue_idx(i + 2, ovr)

            @pl.when(i + 1 < C)
            def _():
                wait_idx(i + 1, nxt)
                issue_gather(i + 1, nxt)

            pltpu.make_async_copy(htab.at[gidx_ring[b]], rows_v,
                                  sem_rows[b]).wait()
            for g in range(K // 16):
                wg = w_buf[pl.ds(g * 16, 16)]
                for e16 in range(16):
                    wv = jnp.full((16,), wg[e16], jnp.float32)
                    e = g * 16 + e16
                    for j in range(HIDDEN // 16):
                        sl2 = pl.ds(j * 16, 16)
                        rows_v[e, sl2] = rows_v[e, sl2] * wv
            pltpu.async_copy(rows_v, out_sh.at[dst_v], sem_scat[b],
                             add=True)

        def round_(r, carry):
            do_chunk(3 * r, 0)
            do_chunk(3 * r + 1, 1)
            do_chunk(3 * r + 2, 2)
            return carry

        lax.fori_loop(0, C // 3, round_, 0)
        fb = (C - 1) % 3
        pltpu.make_async_copy(rows_ring[fb], out_sh.at[dst_ring[fb]],
                              sem_scat[fb]).wait()
        pltpu.sync_copy(denom_v,
                        den_hbm.at[pl.ds((c * NT + s) * NPAD, NA)])
        plsc.subcore_barrier()
        pltpu.sync_copy(out_sh.at[pl.ds(s * RA, RA)],
                        out_hbm.at[pl.ds(c * NPAD + s * RA, RA)])

    return sc_pass


_make_sc_pass = functools.lru_cache(maxsize=None)(_make_sc_pass)


def kernel(x, edge_index, batch, resource_features,
           W1, att_src1, att_dst1, b1,
           W2, att_src2, att_dst2, b2,
           Wr, br, gamma, beta):
    n = x.shape[0]
    loops = jnp.arange(n, dtype=jnp.int32)
    fill = jnp.full((EPAD - E_LOOPS,), n, jnp.int32)
    src = jnp.concatenate([edge_index[0].astype(jnp.int32), loops, fill])
    dst = jnp.concatenate([edge_index[1].astype(jnp.int32), loops, fill])

    x_pad = jnp.zeros((NPAD, D_FEAT), jnp.float32).at[:n].set(x)
    zeros_tab = jnp.zeros((NPAD, HIDDEN), jnp.float32)
    zeros_vec = jnp.zeros((NPAD,), jnp.float32)

    _sc_layer1 = _make_sc_pass(True, CH1)
    _sc_layer2 = _make_sc_pass(False, CH2)

    htab1, a1 = _tc_a(x_pad, W1, att_src1, att_dst1)
    out1, den1 = _sc_layer1(htab1.reshape(HEADS * NPAD, HIDDEN),
                            a1.reshape(2 * HEADS * NPAD), src, dst,
                            zeros_tab, zeros_vec)
    htab2, a2 = _tc_b(out1.reshape(2, NPAD, HIDDEN),
                      den1.reshape(2, NT, NPAD), W2,
                      b1.reshape(1, HEADS * HIDDEN),
                      att_src2, att_dst2)
    out2, den2 = _sc_layer2(htab2, a2.reshape(2 * NPAD), src, dst,
                            zeros_tab, zeros_vec)
    return _tc_c(out2.reshape(2, NPAD, HIDDEN), den2.reshape(2, NT, NPAD),
                 gamma.reshape(1, HIDDEN), beta.reshape(1, HIDDEN),
                 b2.reshape(1, HIDDEN), resource_features, Wr,
                 br.reshape(1, HIDDEN))
